# baseline (speedup 1.0000x reference)
import functools

import jax
import jax.numpy as jnp
from jax import lax
from jax.experimental import pallas as pl
from jax.experimental.pallas import tpu as pltpu
from jax.experimental.pallas import tpu_sc as plsc

B = 512
NUM_COLS = 8
CAT_COLS = 8
NCOLS = NUM_COLS + CAT_COLS
HIDDEN = 64
K = 8
VOCAB = 100000
OUT_DIM = 2
EPS = 1e-5

_ROWS = B * CAT_COLS
_NC, _NS = 2, 16
_NW = _NC * _NS
_BPW = _ROWS // _NW


def _sc_gather(table3, local_idx):
    mesh = plsc.VectorSubcoreMesh(core_axis_name="c", subcore_axis_name="s")

    @functools.partial(
        pl.kernel,
        mesh=mesh,
        out_type=jax.ShapeDtypeStruct((_ROWS, HIDDEN), jnp.float32),
        compiler_params=pltpu.CompilerParams(use_tc_tiling_on_sc=False),
        scratch_types=[
            pltpu.VMEM((_BPW,), jnp.int32),
            pltpu.VMEM((_BPW, HIDDEN), jnp.float32),
            pltpu.SemaphoreType.DMA,
        ],
    )
    def gather(table_hbm, idx_hbm, out_hbm, idx_v, rows_v, sem):
        wid = lax.axis_index("s") * _NC + lax.axis_index("c")
        base = wid * _BPW
        tbl = wid // (B // _BPW)
        pltpu.sync_copy(idx_hbm.at[pl.ds(base, _BPW)], idx_v)
        pltpu.async_copy(table_hbm.at[tbl].at[idx_v], rows_v, sem).wait()
        pltpu.sync_copy(rows_v, out_hbm.at[pl.ds(base, _BPW)])

    return gather(table3, local_idx)


def _tc_body(nd_ref, cat_ref, numw_ref, numb_ref, fiw1_ref, fib1_ref,
             filng_ref, filnb_ref, fiw2_ref, fib2_ref, g1w_ref, g1b_ref,
             g2w_ref, g2b_ref, pw1_ref, pb1_ref, plng_ref, plnb_ref,
             pw2_ref, pb2_ref, out_ref):
    f32 = jnp.float32
    mm = functools.partial(
        lax.dot_general, dimension_numbers=(((1,), (0,)), ((), ())),
        preferred_element_type=f32)

    l8 = lax.broadcasted_iota(jnp.int32, (B, NUM_COLS), 1)
    l16 = lax.broadcasted_iota(jnp.int32, (B, NCOLS), 1)

    def col(m, iota, i):
        return jnp.sum(jnp.where(iota == i, m, 0.0), axis=1, keepdims=True)

    nd = nd_ref[...]
    numw = numw_ref[...]
    numb = numb_ref[...]

    raw = [jnp.maximum(col(nd, l8, i) * numw[i:i + 1, :] + numb[i:i + 1, :],
                       0.0) for i in range(NUM_COLS)]
    tot = NUM_COLS * B * HIDDEN
    m = sum(jnp.sum(b) for b in raw) / tot
    v = sum(jnp.sum((b - m) ** 2) for b in raw) / tot
    fe_num = [(b - m) / jnp.sqrt(v + EPS) for b in raw]

    cat = [cat_ref[i] for i in range(CAT_COLS)]
    m = sum(jnp.sum(b) for b in cat) / tot
    v = sum(jnp.sum((b - m) ** 2) for b in cat) / tot
    fe_cat = [(b - m) / jnp.sqrt(v + EPS) for b in cat]

    fe = fe_num + fe_cat

    fiw1 = fiw1_ref[...]
    fib1 = fib1_ref[...]
    filng = filng_ref[...]
    filnb = filnb_ref[...]
    fiw2 = fiw2_ref[...]
    fib2 = fib2_ref[...]
    fi2d = jnp.zeros((B, NCOLS), f32)
    for t in range(NCOLS):
        h = jnp.maximum(mm(fe[t], fiw1) + fib1, 0.0)
        hm = jnp.mean(h, axis=1, keepdims=True)
        hv = jnp.mean((h - hm) ** 2, axis=1, keepdims=True)
        h = (h - hm) / jnp.sqrt(hv + EPS) * filng + filnb
        fic = jnp.sum(h * fiw2, axis=1, keepdims=True) + fib2
        fi2d = jnp.where(l16 == t, fic, fi2d)
    mf = jnp.mean(fi2d)
    vf_ = jnp.mean((fi2d - mf) ** 2)
    fi2d = (fi2d - mf) / jnp.sqrt(vf_ + EPS)

    fcols = [col(fi2d, l16, t) for t in range(NCOLS)]
    fesc = [fe[t] * fcols[t] for t in range(NCOLS)]

    sels = []
    mrow = jnp.full((B, 1), -jnp.inf, f32)
    for t in range(NCOLS):
        ft = fcols[t]
        rank = jnp.sum((fi2d > ft).astype(f32), axis=1, keepdims=True)
        if t:
            rank = rank + jnp.sum(((fi2d == ft) & (l16 < t)).astype(f32),
                                  axis=1, keepdims=True)
        sel = rank < K
        sels.append(sel)
        mrow = jnp.maximum(mrow, jnp.where(sel, ft, -jnp.inf))
    exps = [jnp.where(sels[t], jnp.exp(fcols[t] - mrow), 0.0)
            for t in range(NCOLS)]
    ssum = functools.reduce(lambda a, b: a + b, exps)
    impc = [e / ssum for e in exps]
    imp2d = jnp.zeros((B, NCOLS), f32)
    for t in range(NCOLS):
        imp2d = jnp.where(l16 == t, impc[t], imp2d)

    g1w = g1w_ref[...]
    h1 = None
    for t in range(NCOLS):
        term = mm(fesc[t], g1w[t * HIDDEN:(t + 1) * HIDDEN, :])
        h1 = term if h1 is None else h1 + term
    g1b = g1b_ref[...]
    g2w = g2w_ref[...]
    g2b = g2b_ref[...]

    ri = lax.broadcasted_iota(jnp.int32, (B, B), 0)
    ci = lax.broadcasted_iota(jnp.int32, (B, B), 1)
    offdiag = ri != ci

    xall = []
    vfs = []
    for half in range(0, NCOLS, 8):
        ts = list(range(half, half + 8))
        vfs_l = [(impc[t] > 0.0).astype(f32) for t in ts]
        subs = [jnp.where(l16 == t, 0.0, imp2d) * vf
                for t, vf in zip(ts, vfs_l)]
        exs = []
        for sub in subs:
            adj = lax.dot_general(sub, sub, (((1,), (1,)), ((), ())),
                                  preferred_element_type=f32)
            em = (adj != 0.0) & offdiag
            exs.append(jnp.where(em, jnp.exp(adj), 0.0))
        ss = [jnp.sum(e) for e in exs]
        rss = [jnp.sum(e, axis=1, keepdims=True) for e in exs]
        cnts = [jnp.sum(vf) * HIDDEN for vf in vfs_l]
        dinvs, diss = [], []
        for s, rs, vf in zip(ss, rss, vfs_l):
            inv_s = jnp.where(s > 0, 1.0 / s, 0.0)
            deg = rs * inv_s + vf
            dinv = jnp.where(deg > 0, 1.0 / jnp.sqrt(deg), 0.0)
            dinvs.append(dinv)
            diss.append(dinv * inv_s)

        xs = [jnp.maximum(dis * mm(e, dinv * h1) + (dinv * dinv) * h1 + g1b,
                          0.0)
              for e, dinv, dis in zip(exs, dinvs, diss)]
        lms = [jnp.sum(x * vf) / c for x, vf, c in zip(xs, vfs_l, cnts)]
        lvs = [jnp.sum(((x - lm) ** 2) * vf) / c
               for x, lm, vf, c in zip(xs, lms, vfs_l, cnts)]
        xs = [(x - lm) / jnp.sqrt(lv + EPS)
              for x, lm, lv in zip(xs, lms, lvs)]

        h2s = [mm(x, g2w) for x in xs]
        xs = [jnp.maximum(dis * mm(e, dinv * h2) + (dinv * dinv) * h2 + g2b,
                          0.0)
              for e, h2, dinv, dis in zip(exs, h2s, dinvs, diss)]
        lms = [jnp.sum(x * vf) / c for x, vf, c in zip(xs, vfs_l, cnts)]
        lvs = [jnp.sum(((x - lm) ** 2) * vf) / c
               for x, lm, vf, c in zip(xs, lms, vfs_l, cnts)]
        xs = [(x - lm) / jnp.sqrt(lv + EPS)
              for x, lm, lv in zip(xs, lms, lvs)]
        xall.extend(xs)
        vfs.extend(vfs_l)

    poss = []
    pos = jnp.zeros((B, 1), f32)
    for t in range(NCOLS):
        poss.append(pos)
        pos = pos + vfs[t]
    nvalid = pos

    pw1 = pw1_ref[...]
    acc = None
    for k in range(K):
        kf = jnp.float32(k)
        slot = jnp.where(nvalid <= kf, xall[NCOLS - 1], 0.0)
        for t in range(NCOLS):
            mask = vfs[t] * (poss[t] == kf).astype(f32)
            slot = slot + mask * xall[t]
        term = mm(slot, pw1[k * HIDDEN:(k + 1) * HIDDEN, :])
        acc = term if acc is None else acc + term
    for c in range(NCOLS):
        acc = acc + mm(fesc[c], pw1[(K + c) * HIDDEN:(K + c + 1) * HIDDEN, :])

    h = jnp.maximum(acc + pb1_ref[...], 0.0)
    hm = jnp.mean(h, axis=1, keepdims=True)
    hv = jnp.mean((h - hm) ** 2, axis=1, keepdims=True)
    h = (h - hm) / jnp.sqrt(hv + EPS) * plng_ref[...] + plnb_ref[...]
    out_ref[...] = mm(h, pw2_ref[...]) + pb2_ref[...]


def _prep(input_data, num_W, num_b, fi_b1, fi_ln_g, fi_ln_b, fi_W2, fi_b2,
          gcn1_b, gcn2_b, pred_b1, pred_ln_g, pred_ln_b, pred_b2):
    nd = input_data[:, :NUM_COLS]
    return (
        nd,
        num_W.reshape(NUM_COLS, HIDDEN),
        num_b,
        fi_b1.reshape(1, HIDDEN),
        fi_ln_g.reshape(1, HIDDEN),
        fi_ln_b.reshape(1, HIDDEN),
        fi_W2.reshape(1, HIDDEN),
        fi_b2.reshape(1, 1),
        gcn1_b.reshape(1, HIDDEN),
        gcn2_b.reshape(1, HIDDEN),
        pred_b1.reshape(1, HIDDEN),
        pred_ln_g.reshape(1, HIDDEN),
        pred_ln_b.reshape(1, HIDDEN),
        pred_b2.reshape(1, OUT_DIM),
    )


def kernel(input_data, num_W, num_b, cat_tables, fi_W1, fi_b1, fi_ln_g,
           fi_ln_b, fi_W2, fi_b2, gcn1_W, gcn1_b, gcn2_W, gcn2_b, pred_W1,
           pred_b1, pred_ln_g, pred_ln_b, pred_W2, pred_b2):
    cat_idx = input_data[:, NUM_COLS:].astype(jnp.int32)
    local_idx = cat_idx.T.reshape(-1)
    rows = _sc_gather(cat_tables, local_idx)
    cat_blocks = rows.reshape(CAT_COLS, B, HIDDEN)

    (nd, numw, numb, fib1, filng, filnb, fiw2, fib2, g1b, g2b, pb1, plng,
     plnb, pb2) = _prep(input_data, num_W, num_b, fi_b1, fi_ln_g, fi_ln_b,
                        fi_W2, fi_b2, gcn1_b, gcn2_b, pred_b1, pred_ln_g,
                        pred_ln_b, pred_b2)

    return pl.pallas_call(
        _tc_body,
        out_shape=jax.ShapeDtypeStruct((B, OUT_DIM), jnp.float32),
    )(nd, cat_blocks, numw, numb, fi_W1, fib1, filng, filnb, fiw2, fib2,
      gcn1_W, g1b, gcn2_W, g2b, pred_W1, pb1, plng, plnb, pred_W2, pb2)


# speedup vs baseline: 1.4489x; 1.4489x over previous
import functools

import jax
import jax.numpy as jnp
from jax import lax
from jax.experimental import pallas as pl
from jax.experimental.pallas import tpu as pltpu
from jax.experimental.pallas import tpu_sc as plsc

B = 512
NUM_COLS = 8
CAT_COLS = 8
NCOLS = NUM_COLS + CAT_COLS
HIDDEN = 64
K = 8
VOCAB = 100000
OUT_DIM = 2
EPS = 1e-5

_ROWS = B * CAT_COLS
_NC, _NS = 2, 16
_NW = _NC * _NS
_BPW = _ROWS // _NW


_HALF = _BPW // 2


def _sc_gather(table3, local_idx):
    mesh = plsc.VectorSubcoreMesh(core_axis_name="c", subcore_axis_name="s")

    @functools.partial(
        pl.kernel,
        mesh=mesh,
        out_type=jax.ShapeDtypeStruct((_ROWS, 8, HIDDEN), jnp.float32),
        scratch_types=[
            pltpu.VMEM((_BPW,), jnp.int32),
            pltpu.VMEM((_HALF, 8, HIDDEN), jnp.float32),
            pltpu.SemaphoreType.DMA,
        ],
    )
    def gather(table_hbm, idx_hbm, out_hbm, idx_v, grp_v, sem):
        wid = lax.axis_index("s") * _NC + lax.axis_index("c")
        base = wid * _BPW
        tbl = wid // (B // _BPW)
        pltpu.sync_copy(idx_hbm.at[pl.ds(base, _BPW)], idx_v)
        for h in range(2):
            copies = []
            for c in range(_HALF // 16):
                vec = idx_v[pl.ds(h * _HALF + c * 16, 16)]
                for lane in range(16):
                    r = vec[lane]
                    g = pl.multiple_of((r // 8) * 8, 8)
                    cp = pltpu.make_async_copy(
                        table_hbm.at[tbl, pl.ds(g, 8)],
                        grp_v.at[c * 16 + lane], sem)
                    cp.start()
                    copies.append(cp)
            for cp in copies:
                cp.wait()
            pltpu.sync_copy(grp_v,
                            out_hbm.at[pl.ds(base + h * _HALF, _HALF)])

    return gather(table3, local_idx)


def _tc_body(nd_ref, grp_ref, off_ref, numw_ref, numb_ref, fiw1_ref, fib1_ref,
             filng_ref, filnb_ref, fiw2_ref, fib2_ref, g1w_ref, g1b_ref,
             g2w_ref, g2b_ref, pw1_ref, pb1_ref, plng_ref, plnb_ref,
             pw2_ref, pb2_ref, out_ref):
    f32 = jnp.float32
    mm = functools.partial(
        lax.dot_general, dimension_numbers=(((1,), (0,)), ((), ())),
        preferred_element_type=f32)

    l8 = lax.broadcasted_iota(jnp.int32, (B, NUM_COLS), 1)
    l16 = lax.broadcasted_iota(jnp.int32, (B, NCOLS), 1)

    def col(m, iota, i):
        return jnp.sum(jnp.where(iota == i, m, 0.0), axis=1, keepdims=True)

    nd = nd_ref[...]
    numw = numw_ref[...]
    numb = numb_ref[...]

    raw = [jnp.maximum(col(nd, l8, i) * numw[i:i + 1, :] + numb[i:i + 1, :],
                       0.0) for i in range(NUM_COLS)]
    tot = NUM_COLS * B * HIDDEN
    m = sum(jnp.sum(b) for b in raw) / tot
    v = sum(jnp.sum((b - m) ** 2) for b in raw) / tot
    fe_num = [(b - m) / jnp.sqrt(v + EPS) for b in raw]

    offm = off_ref[...]
    cat = []
    for i in range(CAT_COLS):
        om = col(offm, l8, i)
        blk = jnp.zeros((B, HIDDEN), f32)
        for o in range(8):
            blk = blk + (om == o).astype(f32) * grp_ref[i, :, o, :]
        cat.append(blk)
    m = sum(jnp.sum(b) for b in cat) / tot
    v = sum(jnp.sum((b - m) ** 2) for b in cat) / tot
    fe_cat = [(b - m) / jnp.sqrt(v + EPS) for b in cat]

    fe = fe_num + fe_cat

    fiw1 = fiw1_ref[...]
    fib1 = fib1_ref[...]
    filng = filng_ref[...]
    filnb = filnb_ref[...]
    fiw2 = fiw2_ref[...]
    fib2 = fib2_ref[...]
    fi2d = jnp.zeros((B, NCOLS), f32)
    for t in range(NCOLS):
        h = jnp.maximum(mm(fe[t], fiw1) + fib1, 0.0)
        hm = jnp.mean(h, axis=1, keepdims=True)
        hv = jnp.mean((h - hm) ** 2, axis=1, keepdims=True)
        h = (h - hm) / jnp.sqrt(hv + EPS) * filng + filnb
        fic = jnp.sum(h * fiw2, axis=1, keepdims=True) + fib2
        fi2d = jnp.where(l16 == t, fic, fi2d)
    mf = jnp.mean(fi2d)
    vf_ = jnp.mean((fi2d - mf) ** 2)
    fi2d = (fi2d - mf) / jnp.sqrt(vf_ + EPS)

    fcols = [col(fi2d, l16, t) for t in range(NCOLS)]
    fesc = [fe[t] * fcols[t] for t in range(NCOLS)]

    sels = []
    mrow = jnp.full((B, 1), -jnp.inf, f32)
    for t in range(NCOLS):
        ft = fcols[t]
        rank = jnp.sum((fi2d > ft).astype(f32), axis=1, keepdims=True)
        if t:
            rank = rank + jnp.sum(((fi2d == ft) & (l16 < t)).astype(f32),
                                  axis=1, keepdims=True)
        sel = rank < K
        sels.append(sel)
        mrow = jnp.maximum(mrow, jnp.where(sel, ft, -jnp.inf))
    exps = [jnp.where(sels[t], jnp.exp(fcols[t] - mrow), 0.0)
            for t in range(NCOLS)]
    ssum = functools.reduce(lambda a, b: a + b, exps)
    impc = [e / ssum for e in exps]
    imp2d = jnp.zeros((B, NCOLS), f32)
    for t in range(NCOLS):
        imp2d = jnp.where(l16 == t, impc[t], imp2d)

    g1w = g1w_ref[...]
    h1 = None
    for t in range(NCOLS):
        term = mm(fesc[t], g1w[t * HIDDEN:(t + 1) * HIDDEN, :])
        h1 = term if h1 is None else h1 + term
    g1b = g1b_ref[...]
    g2w = g2w_ref[...]
    g2b = g2b_ref[...]

    ri = lax.broadcasted_iota(jnp.int32, (B, B), 0)
    ci = lax.broadcasted_iota(jnp.int32, (B, B), 1)
    offdiag = ri != ci

    xall = []
    vfs = []
    for half in range(0, NCOLS, 8):
        ts = list(range(half, half + 8))
        vfs_l = [(impc[t] > 0.0).astype(f32) for t in ts]
        subs = [jnp.where(l16 == t, 0.0, imp2d) * vf
                for t, vf in zip(ts, vfs_l)]
        exs = []
        for sub in subs:
            adj = lax.dot_general(sub, sub, (((1,), (1,)), ((), ())),
                                  preferred_element_type=f32)
            em = (adj != 0.0) & offdiag
            exs.append(jnp.where(em, jnp.exp(adj), 0.0))
        ss = [jnp.sum(e) for e in exs]
        rss = [jnp.sum(e, axis=1, keepdims=True) for e in exs]
        cnts = [jnp.sum(vf) * HIDDEN for vf in vfs_l]
        dinvs, diss = [], []
        for s, rs, vf in zip(ss, rss, vfs_l):
            inv_s = jnp.where(s > 0, 1.0 / s, 0.0)
            deg = rs * inv_s + vf
            dinv = jnp.where(deg > 0, 1.0 / jnp.sqrt(deg), 0.0)
            dinvs.append(dinv)
            diss.append(dinv * inv_s)

        xs = [jnp.maximum(dis * mm(e, dinv * h1) + (dinv * dinv) * h1 + g1b,
                          0.0)
              for e, dinv, dis in zip(exs, dinvs, diss)]
        lms = [jnp.sum(x * vf) / c for x, vf, c in zip(xs, vfs_l, cnts)]
        lvs = [jnp.sum(((x - lm) ** 2) * vf) / c
               for x, lm, vf, c in zip(xs, lms, vfs_l, cnts)]
        xs = [(x - lm) / jnp.sqrt(lv + EPS)
              for x, lm, lv in zip(xs, lms, lvs)]

        h2s = [mm(x, g2w) for x in xs]
        xs = [jnp.maximum(dis * mm(e, dinv * h2) + (dinv * dinv) * h2 + g2b,
                          0.0)
              for e, h2, dinv, dis in zip(exs, h2s, dinvs, diss)]
        lms = [jnp.sum(x * vf) / c for x, vf, c in zip(xs, vfs_l, cnts)]
        lvs = [jnp.sum(((x - lm) ** 2) * vf) / c
               for x, lm, vf, c in zip(xs, lms, vfs_l, cnts)]
        xs = [(x - lm) / jnp.sqrt(lv + EPS)
              for x, lm, lv in zip(xs, lms, lvs)]
        xall.extend(xs)
        vfs.extend(vfs_l)

    poss = []
    pos = jnp.zeros((B, 1), f32)
    for t in range(NCOLS):
        poss.append(pos)
        pos = pos + vfs[t]
    nvalid = pos

    pw1 = pw1_ref[...]
    acc = None
    for k in range(K):
        kf = jnp.float32(k)
        slot = jnp.where(nvalid <= kf, xall[NCOLS - 1], 0.0)
        for t in range(NCOLS):
            mask = vfs[t] * (poss[t] == kf).astype(f32)
            slot = slot + mask * xall[t]
        term = mm(slot, pw1[k * HIDDEN:(k + 1) * HIDDEN, :])
        acc = term if acc is None else acc + term
    for c in range(NCOLS):
        acc = acc + mm(fesc[c], pw1[(K + c) * HIDDEN:(K + c + 1) * HIDDEN, :])

    h = jnp.maximum(acc + pb1_ref[...], 0.0)
    hm = jnp.mean(h, axis=1, keepdims=True)
    hv = jnp.mean((h - hm) ** 2, axis=1, keepdims=True)
    h = (h - hm) / jnp.sqrt(hv + EPS) * plng_ref[...] + plnb_ref[...]
    out_ref[...] = mm(h, pw2_ref[...]) + pb2_ref[...]


def _prep(input_data, num_W, num_b, fi_b1, fi_ln_g, fi_ln_b, fi_W2, fi_b2,
          gcn1_b, gcn2_b, pred_b1, pred_ln_g, pred_ln_b, pred_b2):
    nd = input_data[:, :NUM_COLS]
    return (
        nd,
        num_W.reshape(NUM_COLS, HIDDEN),
        num_b,
        fi_b1.reshape(1, HIDDEN),
        fi_ln_g.reshape(1, HIDDEN),
        fi_ln_b.reshape(1, HIDDEN),
        fi_W2.reshape(1, HIDDEN),
        fi_b2.reshape(1, 1),
        gcn1_b.reshape(1, HIDDEN),
        gcn2_b.reshape(1, HIDDEN),
        pred_b1.reshape(1, HIDDEN),
        pred_ln_g.reshape(1, HIDDEN),
        pred_ln_b.reshape(1, HIDDEN),
        pred_b2.reshape(1, OUT_DIM),
    )


def kernel(input_data, num_W, num_b, cat_tables, fi_W1, fi_b1, fi_ln_g,
           fi_ln_b, fi_W2, fi_b2, gcn1_W, gcn1_b, gcn2_W, gcn2_b, pred_W1,
           pred_b1, pred_ln_g, pred_ln_b, pred_W2, pred_b2):
    cat_idx = input_data[:, NUM_COLS:].astype(jnp.int32)
    local_idx = cat_idx.T.reshape(-1)
    sub_off = (cat_idx % 8).astype(jnp.float32)
    groups = _sc_gather(cat_tables, local_idx)
    grp_blocks = groups.reshape(CAT_COLS, B, 8, HIDDEN)

    (nd, numw, numb, fib1, filng, filnb, fiw2, fib2, g1b, g2b, pb1, plng,
     plnb, pb2) = _prep(input_data, num_W, num_b, fi_b1, fi_ln_g, fi_ln_b,
                        fi_W2, fi_b2, gcn1_b, gcn2_b, pred_b1, pred_ln_g,
                        pred_ln_b, pred_b2)

    return pl.pallas_call(
        _tc_body,
        out_shape=jax.ShapeDtypeStruct((B, OUT_DIM), jnp.float32),
    )(nd, grp_blocks, sub_off, numw, numb, fi_W1, fib1, filng, filnb, fiw2, fib2,
      gcn1_W, g1b, gcn2_W, g2b, pred_W1, pb1, plng, plnb, pred_W2, pb2)
